# baseline (speedup 1.0000x reference)
import functools

import jax
import jax.numpy as jnp
from jax import lax
from jax.experimental import pallas as pl
from jax.experimental.pallas import tpu as pltpu
from jax.experimental.pallas import tpu_sc as plsc

N = 10000
E = 160000
D = 128

NC, NS = 2, 16
NW = NC * NS
E4 = 163840
CHUNK = 512
NCHUNKS = E4 // CHUNK
ROWS_PER_CHUNK = CHUNK // 128
CHUNKS0 = 320
T0 = CHUNKS0 // NS
T1 = (NCHUNKS - CHUNKS0) // NS


def _proj_body(feat_ref, w1s_ref, w1d_ref, wfb_ref, p2_ref, q_ref):
    f = feat_ref[...]
    p = jnp.dot(f, w1s_ref[...], preferred_element_type=jnp.float32)
    p2_ref[0] = p + wfb_ref[0:1, :]
    p2_ref[1] = p + wfb_ref[1:2, :]
    q_ref[...] = jnp.dot(f, w1d_ref[...], preferred_element_type=jnp.float32)


def _node_projections(feat, w1s, w1d, wfb):
    bn = 2000
    grid = (N // bn,)
    p2, q = pl.pallas_call(
        _proj_body,
        grid=grid,
        in_specs=[
            pl.BlockSpec((bn, D), lambda i: (i, 0)),
            pl.BlockSpec((D, D), lambda i: (0, 0)),
            pl.BlockSpec((D, D), lambda i: (0, 0)),
            pl.BlockSpec((2, D), lambda i: (0, 0)),
        ],
        out_specs=[
            pl.BlockSpec((2, bn, D), lambda i: (0, i, 0)),
            pl.BlockSpec((bn, D), lambda i: (i, 0)),
        ],
        out_shape=[
            jax.ShapeDtypeStruct((2, N, D), jnp.float32),
            jax.ShapeDtypeStruct((N, D), jnp.float32),
        ],
    )(feat, w1s, w1d, wfb)
    return p2.reshape(2 * N, D), q


def _sc_gather_body(p2_hbm, q_hbm, src_hbm, dst_hbm, g_hbm,
                    src_v, dst_v, sx_v, buf_v, sem):
    cid = lax.axis_index("c")
    sid = lax.axis_index("s")
    ch0 = jnp.where(cid == 0, sid * T0, CHUNKS0 + sid * T1)
    trips = jnp.where(cid == 0, T0, T1)

    def chunk(ci, _):
        rb = (ch0 + ci) * ROWS_PER_CHUNK
        pltpu.sync_copy(src_hbm.at[pl.ds(rb, ROWS_PER_CHUNK)], src_v)
        pltpu.sync_copy(dst_hbm.at[pl.ds(rb, ROWS_PER_CHUNK)], dst_v)
        for k in range(ROWS_PER_CHUNK):
            for j in range(8):
                s = src_v[k, pl.ds(j * 16, 16)]
                d = dst_v[k, pl.ds(j * 16, 16)]
                sx_v[k, pl.ds(j * 16, 16)] = jnp.where(s == d, s + N, s)
        cps = [
            pltpu.async_copy(p2_hbm.at[sx_v.at[k]],
                             buf_v.at[pl.ds(k * 128, 128)], sem)
            for k in range(ROWS_PER_CHUNK)
        ]
        for cp in cps:
            cp.wait()
        cps = [
            pltpu.async_copy(q_hbm.at[dst_v.at[k]],
                             buf_v.at[pl.ds(k * 128, 128)], sem, add=True)
            for k in range(ROWS_PER_CHUNK)
        ]
        for cp in cps:
            cp.wait()
        pltpu.sync_copy(buf_v, g_hbm.at[pl.ds(rb * 128, CHUNK)])
        return 0

    lax.fori_loop(0, trips, chunk, 0)


def _sc_gather(p2, q, src2d, dst2d):
    mesh = plsc.VectorSubcoreMesh(
        core_axis_name="c", subcore_axis_name="s",
        num_cores=NC, num_subcores=NS)
    fn = pl.kernel(
        _sc_gather_body,
        out_type=jax.ShapeDtypeStruct((E4, D), jnp.float32),
        mesh=mesh,
        scratch_types=[
            pltpu.VMEM((ROWS_PER_CHUNK, 128), jnp.int32),
            pltpu.VMEM((ROWS_PER_CHUNK, 128), jnp.int32),
            pltpu.VMEM((ROWS_PER_CHUNK, 128), jnp.int32),
            pltpu.VMEM((CHUNK, D), jnp.float32),
            pltpu.SemaphoreType.DMA,
        ],
    )
    return fn(p2, q, src2d, dst2d)


def _head_body(g_ref, wcat_ref, b2_ref, loc_ref, scale_ref):
    g = g_ref[...]
    h = g * (1.0 / (1.0 + jnp.exp(-g)))
    o = jnp.dot(h, wcat_ref[...], preferred_element_type=jnp.float32)
    o = o + b2_ref[...]
    loc_ref[...] = o[:, :D]
    scale_ref[...] = jnp.exp(o[:, D:])


def _heads(g, wcat, b2):
    be = 1280
    grid = (E // be,)
    return pl.pallas_call(
        _head_body,
        grid=grid,
        in_specs=[
            pl.BlockSpec((be, D), lambda i: (i, 0)),
            pl.BlockSpec((D, 2 * D), lambda i: (0, 0)),
            pl.BlockSpec((1, 2 * D), lambda i: (0, 0)),
        ],
        out_specs=[
            pl.BlockSpec((be, D), lambda i: (i, 0)),
            pl.BlockSpec((be, D), lambda i: (i, 0)),
        ],
        out_shape=[
            jax.ShapeDtypeStruct((E, D), jnp.float32),
            jax.ShapeDtypeStruct((E, D), jnp.float32),
        ],
        compiler_params=pltpu.CompilerParams(
            dimension_semantics=("arbitrary",)),
    )(g, wcat, b2)


def kernel(feat, edge_index, W1, b1, W_loc, b_loc, W_ls, b_ls):
    src = edge_index[0].astype(jnp.int32)
    dst = edge_index[1].astype(jnp.int32)
    src2d = jnp.pad(src, (0, E4 - E)).reshape(E4 // 128, 128)
    dst2d = jnp.pad(dst, (0, E4 - E)).reshape(E4 // 128, 128)

    w1s = W1[:D]
    w1d = W1[D:2 * D]
    wfb = jnp.stack([b1, b1 + W1[2 * D]])

    p2, q = _node_projections(feat, w1s, w1d, wfb)
    g = _sc_gather(p2, q, src2d, dst2d)

    wcat = jnp.concatenate([W_loc, W_ls], axis=1)
    b2 = jnp.concatenate([b_loc, b_ls]).reshape(1, 2 * D)
    loc, scale = _heads(g, wcat, b2)
    return (loc, scale)


# speedup vs baseline: 1.5786x; 1.5786x over previous
import functools

import jax
import jax.numpy as jnp
from jax import lax
from jax.experimental import pallas as pl
from jax.experimental.pallas import tpu as pltpu
from jax.experimental.pallas import tpu_sc as plsc

N = 10000
E = 160000
D = 128

NC, NS = 2, 16
NW = NC * NS
E4 = 163840
CHUNK = 128
NCHUNKS = E4 // CHUNK
CHUNKS0 = 640
T0 = CHUNKS0 // NS
T1 = (NCHUNKS - CHUNKS0) // NS
NBUF = 6
GDQ = 2
GDW = 4


def _proj_body(feat_ref, w1s_ref, w1d_ref, wfb_ref, p2_ref, q_ref):
    f = feat_ref[...]
    p = jnp.dot(f, w1s_ref[...], preferred_element_type=jnp.float32)
    p2_ref[0] = p + wfb_ref[0:1, :]
    p2_ref[1] = p + wfb_ref[1:2, :]
    q_ref[...] = jnp.dot(f, w1d_ref[...], preferred_element_type=jnp.float32)


def _node_projections(feat, w1s, w1d, wfb):
    bn = 2000
    grid = (N // bn,)
    p2, q = pl.pallas_call(
        _proj_body,
        grid=grid,
        in_specs=[
            pl.BlockSpec((bn, D), lambda i: (i, 0)),
            pl.BlockSpec((D, D), lambda i: (0, 0)),
            pl.BlockSpec((D, D), lambda i: (0, 0)),
            pl.BlockSpec((2, D), lambda i: (0, 0)),
        ],
        out_specs=[
            pl.BlockSpec((2, bn, D), lambda i: (0, i, 0)),
            pl.BlockSpec((bn, D), lambda i: (i, 0)),
        ],
        out_shape=[
            jax.ShapeDtypeStruct((2, N, D), jnp.float32),
            jax.ShapeDtypeStruct((N, D), jnp.float32),
        ],
    )(feat, w1s, w1d, wfb)
    return p2.reshape(2 * N, D), q


def _sc_gather_body(p2_hbm, q_hbm, src_hbm, dst_hbm, g_hbm,
                    src_v, dst_v, sx_v, buf, semI, semP, semQ, semW):
    cid = lax.axis_index("c")
    sid = lax.axis_index("s")
    ch0 = jnp.where(cid == 0, sid * T0, CHUNKS0 + sid * T1)
    T = jnp.where(cid == 0, T0, T1)

    def fire_idx(c):
        b = lax.rem(c, NBUF)
        pltpu.async_copy(src_hbm.at[ch0 + c], src_v.at[b], semI.at[b])
        pltpu.async_copy(dst_hbm.at[ch0 + c], dst_v.at[b], semI.at[b])

    @pl.when(T > 0)
    def _():
        fire_idx(0)

    def body(t, _):
        b = lax.rem(t, NBUF)

        @pl.when(t + 1 < T)
        def _():
            fire_idx(t + 1)

        @pl.when(t < T)
        def _():
            @pl.when(t >= NBUF)
            def _():
                pltpu.make_async_copy(buf.at[pl.ds(b * 128, 128)],
                                      g_hbm.at[pl.ds(0, 128)],
                                      semW.at[b]).wait()
            pltpu.make_async_copy(src_hbm.at[0], src_v.at[b], semI.at[b]).wait()
            pltpu.make_async_copy(dst_hbm.at[0], dst_v.at[b], semI.at[b]).wait()
            for j in range(8):
                s = src_v[b, pl.ds(j * 16, 16)]
                d = dst_v[b, pl.ds(j * 16, 16)]
                sx_v[b, pl.ds(j * 16, 16)] = jnp.where(s == d, s + N, s)
            pltpu.async_copy(p2_hbm.at[sx_v.at[b]],
                             buf.at[pl.ds(b * 128, 128)], semP.at[b])

        cq = t - GDQ

        @pl.when((cq >= 0) & (cq < T))
        def _():
            bq = lax.rem(cq, NBUF)
            pltpu.make_async_copy(g_hbm.at[pl.ds(0, 128)],
                                  buf.at[pl.ds(bq * 128, 128)],
                                  semP.at[bq]).wait()
            pltpu.async_copy(q_hbm.at[dst_v.at[bq]],
                             buf.at[pl.ds(bq * 128, 128)],
                             semQ.at[bq], add=True)

        cw = t - GDW

        @pl.when((cw >= 0) & (cw < T))
        def _():
            bw = lax.rem(cw, NBUF)
            pltpu.make_async_copy(g_hbm.at[pl.ds(0, 128)],
                                  buf.at[pl.ds(bw * 128, 128)],
                                  semQ.at[bw]).wait()
            pltpu.async_copy(buf.at[pl.ds(bw * 128, 128)],
                             g_hbm.at[pl.ds((ch0 + cw) * 128, 128)],
                             semW.at[bw])

        return 0

    lax.fori_loop(0, T + GDW, body, 0)

    for k in range(NBUF):
        @pl.when(T > k)
        def _(k=k):
            bk = lax.rem(T - 1 - k, NBUF)
            pltpu.make_async_copy(buf.at[pl.ds(bk * 128, 128)],
                                  g_hbm.at[pl.ds(0, 128)],
                                  semW.at[bk]).wait()


def _sc_gather(p2, q, src2d, dst2d):
    mesh = plsc.VectorSubcoreMesh(
        core_axis_name="c", subcore_axis_name="s",
        num_cores=NC, num_subcores=NS)
    fn = pl.kernel(
        _sc_gather_body,
        out_type=jax.ShapeDtypeStruct((E4, D), jnp.float32),
        mesh=mesh,
        scratch_types=[
            pltpu.VMEM((NBUF, 128), jnp.int32),
            pltpu.VMEM((NBUF, 128), jnp.int32),
            pltpu.VMEM((NBUF, 128), jnp.int32),
            pltpu.VMEM((NBUF * 128, D), jnp.float32),
            pltpu.SemaphoreType.DMA((NBUF,)),
            pltpu.SemaphoreType.DMA((NBUF,)),
            pltpu.SemaphoreType.DMA((NBUF,)),
            pltpu.SemaphoreType.DMA((NBUF,)),
        ],
    )
    return fn(p2, q, src2d, dst2d)


def _head_body(g_ref, wcat_ref, b2_ref, loc_ref, scale_ref):
    g = g_ref[...]
    h = g * (1.0 / (1.0 + jnp.exp(-g)))
    o = jnp.dot(h, wcat_ref[...], preferred_element_type=jnp.float32)
    o = o + b2_ref[...]
    loc_ref[...] = o[:, :D]
    scale_ref[...] = jnp.exp(o[:, D:])


def _heads(g, wcat, b2):
    be = 1280
    grid = (E // be,)
    return pl.pallas_call(
        _head_body,
        grid=grid,
        in_specs=[
            pl.BlockSpec((be, D), lambda i: (i, 0)),
            pl.BlockSpec((D, 2 * D), lambda i: (0, 0)),
            pl.BlockSpec((1, 2 * D), lambda i: (0, 0)),
        ],
        out_specs=[
            pl.BlockSpec((be, D), lambda i: (i, 0)),
            pl.BlockSpec((be, D), lambda i: (i, 0)),
        ],
        out_shape=[
            jax.ShapeDtypeStruct((E, D), jnp.float32),
            jax.ShapeDtypeStruct((E, D), jnp.float32),
        ],
        compiler_params=pltpu.CompilerParams(
            dimension_semantics=("arbitrary",)),
    )(g, wcat, b2)


def kernel(feat, edge_index, W1, b1, W_loc, b_loc, W_ls, b_ls):
    src = edge_index[0].astype(jnp.int32)
    dst = edge_index[1].astype(jnp.int32)
    src2d = jnp.pad(src, (0, E4 - E)).reshape(E4 // 128, 128)
    dst2d = jnp.pad(dst, (0, E4 - E)).reshape(E4 // 128, 128)

    w1s = W1[:D]
    w1d = W1[D:2 * D]
    wfb = jnp.stack([b1, b1 + W1[2 * D]])

    p2, q = _node_projections(feat, w1s, w1d, wfb)
    g = _sc_gather(p2, q, src2d, dst2d)

    wcat = jnp.concatenate([W_loc, W_ls], axis=1)
    b2 = jnp.concatenate([b_loc, b_ls]).reshape(1, 2 * D)
    loc, scale = _heads(g, wcat, b2)
    return (loc, scale)


# speedup vs baseline: 1.6042x; 1.0162x over previous
import functools

import jax
import jax.numpy as jnp
from jax import lax
from jax.experimental import pallas as pl
from jax.experimental.pallas import tpu as pltpu
from jax.experimental.pallas import tpu_sc as plsc

N = 10000
E = 160000
D = 128

NC, NS = 2, 16
NW = NC * NS
E4 = 163840
CHUNK = 128
NCHUNKS = E4 // CHUNK
CHUNKS0 = 1008
T0 = CHUNKS0 // NS
T1 = (NCHUNKS - CHUNKS0) // NS
NBUF = 6
GDQ = 2
GDW = 4


def _proj_body(feat_ref, w1s_ref, w1d_ref, wfb_ref, p2_ref, q_ref):
    f = feat_ref[...]
    p = jnp.dot(f, w1s_ref[...], preferred_element_type=jnp.float32)
    p2_ref[0] = p + wfb_ref[0:1, :]
    p2_ref[1] = p + wfb_ref[1:2, :]
    q_ref[...] = jnp.dot(f, w1d_ref[...], preferred_element_type=jnp.float32)


def _node_projections(feat, w1s, w1d, wfb):
    bn = 2000
    grid = (N // bn,)
    p2, q = pl.pallas_call(
        _proj_body,
        grid=grid,
        in_specs=[
            pl.BlockSpec((bn, D), lambda i: (i, 0)),
            pl.BlockSpec((D, D), lambda i: (0, 0)),
            pl.BlockSpec((D, D), lambda i: (0, 0)),
            pl.BlockSpec((2, D), lambda i: (0, 0)),
        ],
        out_specs=[
            pl.BlockSpec((2, bn, D), lambda i: (0, i, 0)),
            pl.BlockSpec((bn, D), lambda i: (i, 0)),
        ],
        out_shape=[
            jax.ShapeDtypeStruct((2, N, D), jnp.float32),
            jax.ShapeDtypeStruct((N, D), jnp.float32),
        ],
    )(feat, w1s, w1d, wfb)
    return p2.reshape(2 * N, D), q


def _sc_gather_body(p2_hbm, q_hbm, src_hbm, dst_hbm, g_hbm,
                    src_v, dst_v, sx_v, buf, semI, semP, semQ, semW):
    cid = lax.axis_index("c")
    sid = lax.axis_index("s")
    ch0 = jnp.where(cid == 0, sid * T0, CHUNKS0 + sid * T1)
    T = jnp.where(cid == 0, T0, T1)

    def fire_idx(c):
        b = lax.rem(c, NBUF)
        pltpu.async_copy(src_hbm.at[ch0 + c], src_v.at[b], semI.at[b])
        pltpu.async_copy(dst_hbm.at[ch0 + c], dst_v.at[b], semI.at[b])

    @pl.when(T > 0)
    def _():
        fire_idx(0)

    def body(t, _):
        b = lax.rem(t, NBUF)

        @pl.when(t + 1 < T)
        def _():
            fire_idx(t + 1)

        @pl.when(t < T)
        def _():
            @pl.when(t >= NBUF)
            def _():
                pltpu.make_async_copy(buf.at[pl.ds(b * 128, 128)],
                                      g_hbm.at[pl.ds(0, 128)],
                                      semW.at[b]).wait()
            pltpu.make_async_copy(src_hbm.at[0], src_v.at[b], semI.at[b]).wait()
            pltpu.make_async_copy(dst_hbm.at[0], dst_v.at[b], semI.at[b]).wait()
            for j in range(8):
                s = src_v[b, pl.ds(j * 16, 16)]
                d = dst_v[b, pl.ds(j * 16, 16)]
                sx_v[b, pl.ds(j * 16, 16)] = jnp.where(s == d, s + N, s)
            pltpu.async_copy(p2_hbm.at[sx_v.at[b]],
                             buf.at[pl.ds(b * 128, 128)], semP.at[b])

        cq = t - GDQ

        @pl.when((cq >= 0) & (cq < T))
        def _():
            bq = lax.rem(cq, NBUF)
            pltpu.make_async_copy(g_hbm.at[pl.ds(0, 128)],
                                  buf.at[pl.ds(bq * 128, 128)],
                                  semP.at[bq]).wait()
            pltpu.async_copy(q_hbm.at[dst_v.at[bq]],
                             buf.at[pl.ds(bq * 128, 128)],
                             semQ.at[bq], add=True)

        cw = t - GDW

        @pl.when((cw >= 0) & (cw < T))
        def _():
            bw = lax.rem(cw, NBUF)
            pltpu.make_async_copy(g_hbm.at[pl.ds(0, 128)],
                                  buf.at[pl.ds(bw * 128, 128)],
                                  semQ.at[bw]).wait()
            pltpu.async_copy(buf.at[pl.ds(bw * 128, 128)],
                             g_hbm.at[pl.ds((ch0 + cw) * 128, 128)],
                             semW.at[bw])

        return 0

    lax.fori_loop(0, T + GDW, body, 0)

    for k in range(NBUF):
        @pl.when(T > k)
        def _(k=k):
            bk = lax.rem(T - 1 - k, NBUF)
            pltpu.make_async_copy(buf.at[pl.ds(bk * 128, 128)],
                                  g_hbm.at[pl.ds(0, 128)],
                                  semW.at[bk]).wait()


def _sc_gather(p2, q, src2d, dst2d):
    mesh = plsc.VectorSubcoreMesh(
        core_axis_name="c", subcore_axis_name="s",
        num_cores=NC, num_subcores=NS)
    fn = pl.kernel(
        _sc_gather_body,
        out_type=jax.ShapeDtypeStruct((E4, D), jnp.float32),
        mesh=mesh,
        scratch_types=[
            pltpu.VMEM((NBUF, 128), jnp.int32),
            pltpu.VMEM((NBUF, 128), jnp.int32),
            pltpu.VMEM((NBUF, 128), jnp.int32),
            pltpu.VMEM((NBUF * 128, D), jnp.float32),
            pltpu.SemaphoreType.DMA((NBUF,)),
            pltpu.SemaphoreType.DMA((NBUF,)),
            pltpu.SemaphoreType.DMA((NBUF,)),
            pltpu.SemaphoreType.DMA((NBUF,)),
        ],
    )
    return fn(p2, q, src2d, dst2d)


def _head_body(g_ref, wcat_ref, b2_ref, loc_ref, scale_ref):
    g = g_ref[...]
    h = g * (1.0 / (1.0 + jnp.exp(-g)))
    o = jnp.dot(h, wcat_ref[...], preferred_element_type=jnp.float32)
    o = o + b2_ref[...]
    loc_ref[...] = o[:, :D]
    scale_ref[...] = jnp.exp(o[:, D:])


def _heads(g, wcat, b2):
    be = 1280
    grid = (E // be,)
    return pl.pallas_call(
        _head_body,
        grid=grid,
        in_specs=[
            pl.BlockSpec((be, D), lambda i: (i, 0)),
            pl.BlockSpec((D, 2 * D), lambda i: (0, 0)),
            pl.BlockSpec((1, 2 * D), lambda i: (0, 0)),
        ],
        out_specs=[
            pl.BlockSpec((be, D), lambda i: (i, 0)),
            pl.BlockSpec((be, D), lambda i: (i, 0)),
        ],
        out_shape=[
            jax.ShapeDtypeStruct((E, D), jnp.float32),
            jax.ShapeDtypeStruct((E, D), jnp.float32),
        ],
        compiler_params=pltpu.CompilerParams(
            dimension_semantics=("arbitrary",)),
    )(g, wcat, b2)


def kernel(feat, edge_index, W1, b1, W_loc, b_loc, W_ls, b_ls):
    src = edge_index[0].astype(jnp.int32)
    dst = edge_index[1].astype(jnp.int32)
    src2d = jnp.pad(src, (0, E4 - E)).reshape(E4 // 128, 128)
    dst2d = jnp.pad(dst, (0, E4 - E)).reshape(E4 // 128, 128)

    w1s = W1[:D]
    w1d = W1[D:2 * D]
    wfb = jnp.stack([b1, b1 + W1[2 * D]])

    p2, q = _node_projections(feat, w1s, w1d, wfb)
    g = _sc_gather(p2, q, src2d, dst2d)

    wcat = jnp.concatenate([W_loc, W_ls], axis=1)
    b2 = jnp.concatenate([b_loc, b_ls]).reshape(1, 2 * D)
    loc, scale = _heads(g, wcat, b2)
    return (loc, scale)
